# baseline (speedup 1.0000x reference)
import jax
import jax.numpy as jnp
from jax.experimental import pallas as pl
from jax.experimental.pallas import tpu as pltpu

_D = 1024
_M = 4096
_K = 5
_LBLK = 512
_MBLK = 1024


def _norm_mem_kernel(mem_ref, memn_ref, memb_ref):
    mem = mem_ref[...]
    ssq = jnp.sum(mem * mem, axis=1, keepdims=True)
    n = jnp.maximum(jnp.sqrt(ssq), 1e-12)
    memn_ref[...] = (mem / n).astype(jnp.bfloat16)
    memb_ref[...] = mem.astype(jnp.bfloat16)


def _lmm_block_kernel(nb, enc_sim_ref, enc_out_ref, memn_ref, memb_ref,
                      out_ref, sim2_ref):
    i = pl.program_id(0)
    wslot = jax.lax.rem(i, 2)
    rslot = 1 - wslot

    enc = enc_sim_ref[...]
    essq = jnp.sum(enc * enc, axis=1, keepdims=True)
    en = (enc / jnp.maximum(jnp.sqrt(essq), 1e-12)).astype(jnp.bfloat16)
    sim2_ref[wslot] = jax.lax.dot_general(
        en, memn_ref[...], (((1,), (1,)), ((), ())),
        preferred_element_type=jnp.float32)

    sim = sim2_ref[rslot]
    neg = jnp.float32(-jnp.inf)
    mx = jnp.max(sim, axis=1, keepdims=True)
    for _ in range(_K - 1):
        mx = jnp.max(jnp.where(sim < mx, sim, neg), axis=1, keepdims=True)

    maskb = (sim >= mx).astype(jnp.bfloat16)
    cnt = jnp.sum(maskb.astype(jnp.float32), axis=1, keepdims=True)
    matched = jax.lax.dot_general(
        maskb, memb_ref[...], (((1,), (0,)), ((), ())),
        preferred_element_type=jnp.float32)
    out_ref[...] = enc_out_ref[...] + matched / cnt


def kernel(encoded, memory):
    B, L, D = encoded.shape
    M = memory.shape[0]
    x2d = encoded.reshape(B * L, D)
    nb = (B * L) // _LBLK

    memn, memb = pl.pallas_call(
        _norm_mem_kernel,
        grid=(M // _MBLK,),
        in_specs=[pl.BlockSpec((_MBLK, D), lambda i: (i, 0))],
        out_specs=[pl.BlockSpec((_MBLK, D), lambda i: (i, 0)),
                   pl.BlockSpec((_MBLK, D), lambda i: (i, 0))],
        out_shape=[jax.ShapeDtypeStruct((M, D), jnp.bfloat16),
                   jax.ShapeDtypeStruct((M, D), jnp.bfloat16)],
    )(memory)

    import functools
    out = pl.pallas_call(
        functools.partial(_lmm_block_kernel, nb),
        grid=(nb + 1,),
        in_specs=[
            pl.BlockSpec((_LBLK, D), lambda i: (jnp.minimum(i, nb - 1), 0)),
            pl.BlockSpec((_LBLK, D), lambda i: (jnp.maximum(i - 1, 0), 0)),
            pl.BlockSpec((M, D), lambda i: (0, 0)),
            pl.BlockSpec((M, D), lambda i: (0, 0)),
        ],
        out_specs=pl.BlockSpec((_LBLK, D), lambda i: (jnp.maximum(i - 1, 0), 0)),
        out_shape=jax.ShapeDtypeStruct((B * L, D), jnp.float32),
        scratch_shapes=[pltpu.VMEM((2, _LBLK, M), jnp.float32)],
        compiler_params=pltpu.CompilerParams(
            vmem_limit_bytes=100 * 1024 * 1024),
    )(x2d, x2d, memn, memb)
    return out.reshape(B, L, D)


# speedup vs baseline: 1.1473x; 1.1473x over previous
import jax
import jax.numpy as jnp
from jax.experimental import pallas as pl
from jax.experimental.pallas import tpu as pltpu

_D = 1024
_M = 4096
_K = 5
_LBLK = 1024
_MBLK = 1024


def _norm_mem_kernel(mem_ref, memn_ref, memb_ref):
    mem = mem_ref[...]
    ssq = jnp.sum(mem * mem, axis=1, keepdims=True)
    n = jnp.maximum(jnp.sqrt(ssq), 1e-12)
    memn_ref[...] = (mem / n).astype(jnp.bfloat16)
    memb_ref[...] = mem.astype(jnp.bfloat16)


def _lmm_block_kernel(enc_ref, memn_ref, memb_ref, out_ref):
    enc = enc_ref[...]
    essq = jnp.sum(enc * enc, axis=1, keepdims=True)
    en = (enc / jnp.maximum(jnp.sqrt(essq), 1e-12)).astype(jnp.bfloat16)

    sim = jax.lax.dot_general(
        en, memn_ref[...], (((1,), (1,)), ((), ())),
        preferred_element_type=jnp.float32)

    neg = jnp.float32(-jnp.inf)
    mx = jnp.max(sim, axis=1, keepdims=True)
    for _ in range(_K - 1):
        mx = jnp.max(jnp.where(sim < mx, sim, neg), axis=1, keepdims=True)

    maskb = (sim >= mx).astype(jnp.bfloat16)
    cnt = jnp.sum(maskb.astype(jnp.float32), axis=1, keepdims=True)
    matched = jax.lax.dot_general(
        maskb, memb_ref[...], (((1,), (0,)), ((), ())),
        preferred_element_type=jnp.float32)
    out_ref[...] = enc + matched / cnt


def kernel(encoded, memory):
    B, L, D = encoded.shape
    M = memory.shape[0]
    x2d = encoded.reshape(B * L, D)
    nb = (B * L) // _LBLK

    memn, memb = pl.pallas_call(
        _norm_mem_kernel,
        grid=(M // _MBLK,),
        in_specs=[pl.BlockSpec((_MBLK, D), lambda i: (i, 0))],
        out_specs=[pl.BlockSpec((_MBLK, D), lambda i: (i, 0)),
                   pl.BlockSpec((_MBLK, D), lambda i: (i, 0))],
        out_shape=[jax.ShapeDtypeStruct((M, D), jnp.bfloat16),
                   jax.ShapeDtypeStruct((M, D), jnp.bfloat16)],
    )(memory)

    out = pl.pallas_call(
        _lmm_block_kernel,
        grid=(nb,),
        in_specs=[
            pl.BlockSpec((_LBLK, D), lambda i: (i, 0)),
            pl.BlockSpec((M, D), lambda i: (0, 0)),
            pl.BlockSpec((M, D), lambda i: (0, 0)),
        ],
        out_specs=pl.BlockSpec((_LBLK, D), lambda i: (i, 0)),
        out_shape=jax.ShapeDtypeStruct((B * L, D), jnp.float32),
        compiler_params=pltpu.CompilerParams(
            vmem_limit_bytes=100 * 1024 * 1024),
    )(x2d, memn, memb)
    return out.reshape(B, L, D)
